# baseline (speedup 1.0000x reference)
import functools

import jax
import jax.numpy as jnp
from jax import lax
from jax.experimental import pallas as pl
from jax.experimental.pallas import tpu as pltpu
from jax.experimental.pallas import tpu_sc as plsc

N = 10000
E = 320000
C = 128
OUT = 128

_NC = 2
_NS = 16
_NW = _NC * _NS
CHUNK = 128
_CH = E // CHUNK
_CHB = _CH // _NW
_CHR = _CH % _NW
_RT = N // _NS
_RZ = (N + 16) // _NS


_NP = N + 16


def _counts_body(e0_hbm, e1_hbm, zc_hbm,
                 csl_hbm, cdl_hbm,
                 e0_v, e1_v, cs_loc, cd_loc):
    cid = lax.axis_index("c")
    sid = lax.axis_index("s")
    wid = sid * _NC + cid
    one16 = jnp.ones((16,), jnp.float32)
    pltpu.sync_copy(zc_hbm, cs_loc)
    pltpu.sync_copy(zc_hbm, cd_loc)
    nch = jnp.where(wid < _CHR, _CHB + 1, _CHB)

    def _chunk(j, carry):
        off = (wid + j * _NW) * CHUNK
        pltpu.sync_copy(e0_hbm.at[pl.ds(off, CHUNK)], e0_v)
        pltpu.sync_copy(e1_hbm.at[pl.ds(off, CHUNK)], e1_v)

        def _cnt(i, c2):
            plsc.addupdate_scatter(cs_loc, [e0_v[pl.ds(i * 16, 16)]], one16)
            plsc.addupdate_scatter(cd_loc, [e1_v[pl.ds(i * 16, 16)]], one16)
            return c2

        lax.fori_loop(0, CHUNK // 16, _cnt, 0)
        return carry

    lax.fori_loop(0, nch, _chunk, 0)
    pltpu.sync_copy(cs_loc, csl_hbm.at[pl.ds(wid * _NP, _NP)])
    pltpu.sync_copy(cd_loc, cdl_hbm.at[pl.ds(wid * _NP, _NP)])


_counts_kernel = functools.partial(
    pl.kernel,
    out_type=[jax.ShapeDtypeStruct((_NW * _NP,), jnp.float32)] * 2,
    mesh=plsc.VectorSubcoreMesh(core_axis_name="c", subcore_axis_name="s"),
    compiler_params=pltpu.CompilerParams(use_tc_tiling_on_sc=False, needs_layout_passes=False),
    scratch_types=[
        pltpu.VMEM((CHUNK,), jnp.int32),
        pltpu.VMEM((CHUNK,), jnp.int32),
        pltpu.VMEM((_NP,), jnp.float32),
        pltpu.VMEM((_NP,), jnp.float32),
    ],
)(_counts_body)


def _agg_body(e0_hbm, e1_hbm, t_hbm, x_hbm, z_hbm,
              a0_hbm, a1_hbm,
              e0_v, e1_v, tgt_v, rows_v, t_v, agg_sh, sem):
    cid = lax.axis_index("c")
    sid = lax.axis_index("s")
    wid = sid * _NC + cid
    pltpu.sync_copy(t_hbm, t_v)
    pltpu.sync_copy(z_hbm, agg_sh.at[pl.ds(sid * _RZ, _RZ)])
    plsc.subcore_barrier()
    nch = jnp.where(wid < _CHR, _CHB + 1, _CHB)

    def _chunk(j, carry):
        off = (wid + j * _NW) * CHUNK
        pltpu.sync_copy(e1_hbm.at[pl.ds(off, CHUNK)], e1_v)
        cp = pltpu.async_copy(x_hbm.at[e1_v], rows_v, sem)
        pltpu.sync_copy(e0_hbm.at[pl.ds(off, CHUNK)], e0_v)

        def _map(i, c2):
            idx16 = e0_v[pl.ds(i * 16, 16)]
            tgt_v[pl.ds(i * 16, 16)] = plsc.load_gather(t_v, [idx16])
            return c2

        lax.fori_loop(0, CHUNK // 16, _map, 0)
        cp.wait()
        pltpu.sync_copy(rows_v, agg_sh.at[tgt_v], add=True)
        return carry

    lax.fori_loop(0, nch, _chunk, 0)
    plsc.subcore_barrier()
    r0 = sid * _RT

    @pl.when(cid == 0)
    def _():
        pltpu.sync_copy(agg_sh.at[pl.ds(r0, _RT)], a0_hbm.at[pl.ds(r0, _RT)])

    @pl.when(cid == 1)
    def _():
        pltpu.sync_copy(agg_sh.at[pl.ds(r0, _RT)], a1_hbm.at[pl.ds(r0, _RT)])


_agg_kernel = functools.partial(
    pl.kernel,
    out_type=[jax.ShapeDtypeStruct((N, C), jnp.float32)] * 2,
    mesh=plsc.VectorSubcoreMesh(core_axis_name="c", subcore_axis_name="s"),
    compiler_params=pltpu.CompilerParams(use_tc_tiling_on_sc=False, needs_layout_passes=False),
    scratch_types=[
        pltpu.VMEM((CHUNK,), jnp.int32),
        pltpu.VMEM((CHUNK,), jnp.int32),
        pltpu.VMEM((CHUNK,), jnp.int32),
        pltpu.VMEM((CHUNK, C), jnp.float32),
        pltpu.VMEM((N,), jnp.int32),
        pltpu.VMEM_SHARED((N + 16, C), jnp.float32),
        pltpu.SemaphoreType.DMA,
    ],
)(_agg_body)


_RB = 1000


def _dense_body(p0, p1, x, cnt, wl, wc, bc, ng, nb, wo, bo, o):
    cntv = cnt[...]
    deg = jnp.maximum(cntv, 1.0)
    agg = (p0[...] + p1[...]) / deg
    pre = (jnp.dot(agg, wl[...], preferred_element_type=jnp.float32)
           + jnp.dot(x[...], wc[...], preferred_element_type=jnp.float32)
           + bc[...])
    h = jnp.maximum(pre, 0.0)
    mu = jnp.mean(h, axis=1, keepdims=True)
    var = jnp.mean((h - mu) * (h - mu), axis=1, keepdims=True)
    ln = (h - mu) * lax.rsqrt(var + 1e-5) * ng[...] + nb[...]
    hsel = jnp.where(cntv > 0.0, ln, x[...])
    o[...] = jnp.dot(hsel, wo[...], preferred_element_type=jnp.float32) + bo[...]


_dense_kernel = pl.pallas_call(
    _dense_body,
    grid=(N // _RB,),
    in_specs=[
        pl.BlockSpec((_RB, C), lambda i: (i, 0)),
        pl.BlockSpec((_RB, C), lambda i: (i, 0)),
        pl.BlockSpec((_RB, C), lambda i: (i, 0)),
        pl.BlockSpec((_RB, 1), lambda i: (i, 0)),
        pl.BlockSpec((C, C), lambda i: (0, 0)),
        pl.BlockSpec((C, C), lambda i: (0, 0)),
        pl.BlockSpec((1, C), lambda i: (0, 0)),
        pl.BlockSpec((1, C), lambda i: (0, 0)),
        pl.BlockSpec((1, C), lambda i: (0, 0)),
        pl.BlockSpec((C, OUT), lambda i: (0, 0)),
        pl.BlockSpec((1, OUT), lambda i: (0, 0)),
    ],
    out_specs=pl.BlockSpec((_RB, OUT), lambda i: (i, 0)),
    out_shape=jax.ShapeDtypeStruct((N, OUT), jnp.float32),
)


def kernel(x_user, x_item, edge_user_item, edge_item_user,
           conv0_Wl, conv0_W0, conv0_W1, conv0_bl, conv0_b0, conv0_b1,
           conv0_gate, norm0_g, norm0_b,
           conv1_Wl, conv1_W0, conv1_W1, conv1_bl, conv1_b0, conv1_b1,
           conv1_gate, norm1_g, norm1_b,
           W_out, b_out):
    e0 = edge_item_user[0]
    e1 = edge_item_user[1]

    zc = jnp.zeros((_NP,), jnp.float32)
    csl, cdl = _counts_kernel(e0, e1, zc)

    cnt2 = jnp.stack([csl, cdl]).reshape(2, _NW, _NP).sum(1)[:, :N]
    cnt_s = cnt2[0]
    cnt_d = cnt2[1]
    pres = cnt2 > 0.0
    ranks = jnp.cumsum(pres.astype(jnp.int32), axis=1) - 1
    ps = pres[0]
    pd = pres[1]
    rank_s = ranks[0]
    rank_d = ranks[1]
    ar = jnp.arange(N, dtype=jnp.int32)
    nor = jnp.full((N,), N, jnp.int32).at[
        jnp.where(pd, rank_d, N)].set(ar, mode="drop")
    tmap = jnp.where(ps, nor[jnp.clip(rank_s, 0, N - 1)],
                     jnp.int32(N)).astype(jnp.int32)

    za = jnp.zeros((_RZ, C), jnp.float32)
    a0, a1 = _agg_kernel(e0, e1, tmap, x_user, za)

    g = jax.nn.sigmoid(conv1_gate)
    wc = (1.0 - g) * conv1_W0 + g * conv1_W1
    bc = conv1_bl + (1.0 - g) * conv1_b0 + g * conv1_b1

    return _dense_kernel(a0, a1, x_user, cnt_d[:, None],
                         conv1_Wl, wc, bc[None], norm1_g[None], norm1_b[None],
                         W_out, b_out[None])
